# Initial kernel scaffold; baseline (speedup 1.0000x reference)
import jax
import jax.numpy as jnp
from jax.experimental import pallas as pl


def kernel(x, edge_index, batch, W1, b1, W2, b2):
    raise NotImplementedError("write your pallas kernel here")


# speedup vs baseline: 54.4647x; 54.4647x over previous
import functools

import jax
import jax.numpy as jnp
from jax import lax
from jax.experimental import pallas as pl
from jax.experimental.pallas import tpu as pltpu
from jax.experimental.pallas import tpu_sc as plsc

N_NODES = 10000
N_PAD = 10240
ROWS_PER_TILE = 640
HID = 16
D_FEAT = 128
NUM_EDGES = 320000
CHUNK = 128
CHUNKS_PER_TILE = 80
E_PAD = 32 * CHUNKS_PER_TILE * CHUNK
HIST_PER_TILE = E_PAD // 16
HIST_CHUNK = 2048
ROWCHUNK = 160
NG = 64


def _mm_body(x_ref, w_ref, o_ref):
    o_ref[...] = jnp.dot(x_ref[...], w_ref[...],
                         preferred_element_type=jnp.float32)


def _matmul(x_pad, W1):
    return pl.pallas_call(
        _mm_body,
        out_shape=jax.ShapeDtypeStruct((N_PAD, HID), jnp.float32),
    )(x_pad, W1)


_MESH = plsc.VectorSubcoreMesh(core_axis_name="c", subcore_axis_name="s")


@functools.partial(
    pl.kernel,
    mesh=_MESH,
    compiler_params=pltpu.CompilerParams(needs_layout_passes=False,
                                         use_tc_tiling_on_sc=False),
    out_type=[
        jax.ShapeDtypeStruct((2, N_PAD, HID), jnp.float32),
        jax.ShapeDtypeStruct((N_PAD,), jnp.float32),
        jax.ShapeDtypeStruct((N_PAD, HID), jnp.float32),
    ],
    scratch_types=[
        pltpu.VMEM((HIST_CHUNK,), jnp.int32),
        pltpu.VMEM((N_PAD,), jnp.float32),
        pltpu.VMEM((ROWS_PER_TILE,), jnp.float32),
        pltpu.VMEM((ROWCHUNK, HID), jnp.float32),
        pltpu.VMEM((CHUNKS_PER_TILE * CHUNK,), jnp.int32),
        pltpu.VMEM((CHUNKS_PER_TILE * CHUNK,), jnp.int32),
        pltpu.VMEM((CHUNK,), jnp.int32),
        pltpu.VMEM((CHUNK,), jnp.int32),
        pltpu.VMEM((CHUNK, HID), jnp.float32),
        pltpu.VMEM_SHARED((16, N_PAD), jnp.float32),
        pltpu.VMEM_SHARED((N_PAD, HID), jnp.float32),
        pltpu.VMEM_SHARED((N_PAD, HID), jnp.float32),
        pltpu.SemaphoreType.DMA,
    ],
)
def _sc_gcn(h_hbm, srcflat_hbm, dstflat_hbm,
            acc_out, dis_out, hp_out,
            dstbuf, degbuf, disbuf, hbuf, srcidx, dstidx, srcchunk, dstchunk,
            msg, stage_sh, h_sh, acc_sh, sem):
    c = lax.axis_index("c")
    s = lax.axis_index("s")
    wid = c * 16 + s
    row0 = s * ROWS_PER_TILE

    zeros16 = jnp.zeros((16,), jnp.float32)
    ones16 = jnp.ones((16,), jnp.float32)

    def _zdeg(i, carry):
        degbuf[pl.ds(i * 16, 16)] = zeros16
        return carry
    lax.fori_loop(0, N_PAD // 16, _zdeg, 0)

    def _zh(i, carry):
        hbuf[i, :] = zeros16
        return carry
    lax.fori_loop(0, ROWCHUNK, _zh, 0)
    for rc in range(ROWS_PER_TILE // ROWCHUNK):
        pltpu.sync_copy(hbuf, acc_sh.at[pl.ds(row0 + rc * ROWCHUNK, ROWCHUNK)])

    for hc in range(HIST_PER_TILE // HIST_CHUNK):
        pltpu.sync_copy(
            dstflat_hbm.at[pl.ds(s * HIST_PER_TILE + hc * HIST_CHUNK,
                                 HIST_CHUNK)],
            dstbuf)

        def _hist(i, carry):
            idx = dstbuf[pl.ds(i * 16, 16)]
            plsc.addupdate_scatter(degbuf, [idx], ones16)
            return carry
        lax.fori_loop(0, HIST_CHUNK // 16, _hist, 0)
    pltpu.sync_copy(degbuf, stage_sh.at[s])
    plsc.subcore_barrier()

    pltpu.sync_copy(stage_sh.at[0, pl.ds(row0, ROWS_PER_TILE)], disbuf)
    for j in range(1, 16):
        pltpu.sync_copy(stage_sh.at[j, pl.ds(row0, ROWS_PER_TILE)],
                        degbuf.at[pl.ds(0, ROWS_PER_TILE)])

        def _acc(k, carry):
            disbuf[pl.ds(k * 16, 16)] = (disbuf[pl.ds(k * 16, 16)]
                                         + degbuf[pl.ds(k * 16, 16)])
            return carry
        lax.fori_loop(0, ROWS_PER_TILE // 16, _acc, 0)

    def _rsqrt(k, carry):
        d = disbuf[pl.ds(k * 16, 16)] + jnp.float32(1.0)
        bits = lax.bitcast_convert_type(d, jnp.int32)
        y = lax.bitcast_convert_type(jnp.int32(0x5F3759DF) - (bits >> 1),
                                     jnp.float32)
        for _ in range(3):
            y = y * (jnp.float32(1.5) - jnp.float32(0.5) * d * y * y)
        disbuf[pl.ds(k * 16, 16)] = y
        return carry
    lax.fori_loop(0, ROWS_PER_TILE // 16, _rsqrt, 0)

    @pl.when(c == 0)
    def _():
        pltpu.sync_copy(disbuf, dis_out.at[pl.ds(row0, ROWS_PER_TILE)])

    for rc in range(ROWS_PER_TILE // ROWCHUNK):
        pltpu.sync_copy(h_hbm.at[pl.ds(row0 + rc * ROWCHUNK, ROWCHUNK)], hbuf)

        def _scale(j, carry):
            sc_vec = plsc.load_gather(
                disbuf, [jnp.full((16,), rc * ROWCHUNK, jnp.int32) + j])
            hbuf[j, :] = hbuf[j, :] * sc_vec
            return carry
        lax.fori_loop(0, ROWCHUNK, _scale, 0)
        pltpu.sync_copy(hbuf, h_sh.at[pl.ds(row0 + rc * ROWCHUNK, ROWCHUNK)])
        pltpu.sync_copy(hbuf, hp_out.at[pl.ds(row0 + rc * ROWCHUNK, ROWCHUNK)])
    plsc.subcore_barrier()

    e_per_tile = CHUNKS_PER_TILE * CHUNK
    pltpu.sync_copy(srcflat_hbm.at[pl.ds(wid * e_per_tile, e_per_tile)],
                    srcidx)
    pltpu.sync_copy(dstflat_hbm.at[pl.ds(wid * e_per_tile, e_per_tile)],
                    dstidx)

    def _edges(j, carry):
        def _fill(k, carry2):
            srcchunk[pl.ds(k * 16, 16)] = srcidx[pl.ds(j * CHUNK + k * 16, 16)]
            dstchunk[pl.ds(k * 16, 16)] = dstidx[pl.ds(j * CHUNK + k * 16, 16)]
            return carry2
        lax.fori_loop(0, CHUNK // 16, _fill, 0)
        pltpu.async_copy(h_sh.at[srcchunk], msg, sem).wait()
        pltpu.sync_copy(msg, acc_sh.at[dstchunk], add=True)
        return carry
    lax.fori_loop(0, CHUNKS_PER_TILE, _edges, 0)
    plsc.subcore_barrier()

    pltpu.sync_copy(acc_sh.at[pl.ds(row0, ROWS_PER_TILE)],
                    acc_out.at[c, pl.ds(row0, ROWS_PER_TILE)])


def _final_body(h_ref, acc_ref, dis_ref, batch_ref, b1_ref, w2_ref, b2_ref,
                o_ref):
    h = h_ref[...]
    acc = acc_ref[0] + acc_ref[1]
    dis = dis_ref[...]
    tmp = dis[:, None] * (dis[:, None] * h + acc) + b1_ref[...][None, :]
    nodes = jnp.maximum(tmp, 0.0)
    b = batch_ref[...]
    onehot = (b[:, None] == lax.broadcasted_iota(jnp.int32, (N_PAD, NG), 1)
              ).astype(jnp.float32)
    counts = jnp.sum(onehot, axis=0)
    sums = lax.dot_general(onehot, nodes, (((0,), (0,)), ((), ())),
                           preferred_element_type=jnp.float32)
    g = sums / jnp.clip(counts, 1.0)[:, None]
    logits = jnp.dot(g, w2_ref[...], preferred_element_type=jnp.float32)
    logits = logits + b2_ref[...][None, :]
    m = jnp.max(logits, axis=1, keepdims=True)
    lse = jnp.log(jnp.sum(jnp.exp(logits - m), axis=1, keepdims=True)) + m
    o_ref[...] = logits - lse


def _final(h, acc, dis, batch_pad, b1, W2, b2):
    return pl.pallas_call(
        _final_body,
        out_shape=jax.ShapeDtypeStruct((NG, 10), jnp.float32),
    )(h, acc, dis, batch_pad, b1, W2, b2)


def kernel(x, edge_index, batch, W1, b1, W2, b2):
    src = edge_index[0].astype(jnp.int32)
    dst = edge_index[1].astype(jnp.int32)
    pad = jnp.full((E_PAD - NUM_EDGES,), N_NODES, jnp.int32)
    src_p = jnp.concatenate([src, pad])
    dst_p = jnp.concatenate([dst, pad])
    x_pad = jnp.concatenate(
        [x, jnp.zeros((N_PAD - N_NODES, D_FEAT), jnp.float32)])
    batch_pad = jnp.concatenate(
        [batch.astype(jnp.int32), jnp.full((N_PAD - N_NODES,), NG, jnp.int32)])

    h = _matmul(x_pad, W1)
    acc, dis, _hp = _sc_gcn(h, src_p, dst_p)
    return _final(h, acc, dis, batch_pad, b1, W2, b2)


# speedup vs baseline: 60.2216x; 1.1057x over previous
import functools

import jax
import jax.numpy as jnp
from jax import lax
from jax.experimental import pallas as pl
from jax.experimental.pallas import tpu as pltpu
from jax.experimental.pallas import tpu_sc as plsc

N_NODES = 10000
N_PAD = 10240
ROWS_PER_TILE = 640
HID = 16
D_FEAT = 128
NUM_EDGES = 320000
CHUNK = 128
CHUNKS_PER_TILE = 80
E_ROWS = 32 * CHUNKS_PER_TILE
E_PAD = E_ROWS * CHUNK
HIST_ROWS_PER_TILE = E_ROWS // 16
HIST_ROWCHUNK = 16
ROWCHUNK = 160
NG = 64


def _mm_body(x_ref, w_ref, o_ref):
    o_ref[...] = jnp.dot(x_ref[...], w_ref[...],
                         preferred_element_type=jnp.float32)


def _matmul(x_pad, W1):
    return pl.pallas_call(
        _mm_body,
        out_shape=jax.ShapeDtypeStruct((N_PAD, HID), jnp.float32),
    )(x_pad, W1)


_MESH = plsc.VectorSubcoreMesh(core_axis_name="c", subcore_axis_name="s")


@functools.partial(
    pl.kernel,
    mesh=_MESH,
    compiler_params=pltpu.CompilerParams(needs_layout_passes=False,
                                         use_tc_tiling_on_sc=False),
    out_type=[
        jax.ShapeDtypeStruct((2, N_PAD, HID), jnp.float32),
        jax.ShapeDtypeStruct((N_PAD,), jnp.float32),
    ],
    scratch_types=[
        pltpu.VMEM((HIST_ROWCHUNK, CHUNK), jnp.int32),
        pltpu.VMEM((N_PAD,), jnp.float32),
        pltpu.VMEM((ROWS_PER_TILE,), jnp.float32),
        pltpu.VMEM((ROWCHUNK, HID), jnp.float32),
        pltpu.VMEM((CHUNKS_PER_TILE, CHUNK), jnp.int32),
        pltpu.VMEM((CHUNKS_PER_TILE, CHUNK), jnp.int32),
        pltpu.VMEM((CHUNK, HID), jnp.float32),
        pltpu.VMEM((CHUNK, HID), jnp.float32),
        pltpu.VMEM_SHARED((16, N_PAD), jnp.float32),
        pltpu.VMEM_SHARED((N_PAD, HID), jnp.float32),
        pltpu.VMEM_SHARED((N_PAD, HID), jnp.float32),
        pltpu.SemaphoreType.DMA,
        pltpu.SemaphoreType.DMA,
    ],
)
def _sc_gcn(h_hbm, src2d_hbm, dst2d_hbm,
            acc_out, dis_out,
            dstbuf, degbuf, disbuf, hbuf, srcidx, dstidx,
            msga, msgb, stage_sh, h_sh, acc_sh, sema, semb):
    c = lax.axis_index("c")
    s = lax.axis_index("s")
    wid = c * 16 + s
    row0 = s * ROWS_PER_TILE

    zeros16 = jnp.zeros((16,), jnp.float32)
    ones16 = jnp.ones((16,), jnp.float32)

    def _zdeg(i, carry):
        degbuf[pl.ds(i * 16, 16)] = zeros16
        return carry
    lax.fori_loop(0, N_PAD // 16, _zdeg, 0)

    def _zh(i, carry):
        hbuf[i, :] = zeros16
        return carry
    lax.fori_loop(0, ROWCHUNK, _zh, 0)
    for rc in range(ROWS_PER_TILE // ROWCHUNK):
        pltpu.sync_copy(hbuf, acc_sh.at[pl.ds(row0 + rc * ROWCHUNK, ROWCHUNK)])

    for hc in range(HIST_ROWS_PER_TILE // HIST_ROWCHUNK):
        pltpu.sync_copy(
            dst2d_hbm.at[pl.ds(s * HIST_ROWS_PER_TILE + hc * HIST_ROWCHUNK,
                               HIST_ROWCHUNK)],
            dstbuf)

        def _histrow(r, carry):
            def _hist(k, carry2):
                idx = dstbuf[r, pl.ds(k * 16, 16)]
                plsc.addupdate_scatter(degbuf, [idx], ones16)
                return carry2
            lax.fori_loop(0, CHUNK // 16, _hist, 0)
            return carry
        lax.fori_loop(0, HIST_ROWCHUNK, _histrow, 0)
    pltpu.sync_copy(degbuf, stage_sh.at[s])
    plsc.subcore_barrier()

    pltpu.sync_copy(stage_sh.at[0, pl.ds(row0, ROWS_PER_TILE)], disbuf)
    for j in range(1, 16):
        pltpu.sync_copy(stage_sh.at[j, pl.ds(row0, ROWS_PER_TILE)],
                        degbuf.at[pl.ds(0, ROWS_PER_TILE)])

        def _acc(k, carry):
            disbuf[pl.ds(k * 16, 16)] = (disbuf[pl.ds(k * 16, 16)]
                                         + degbuf[pl.ds(k * 16, 16)])
            return carry
        lax.fori_loop(0, ROWS_PER_TILE // 16, _acc, 0)

    def _rsqrt(k, carry):
        d = disbuf[pl.ds(k * 16, 16)] + jnp.float32(1.0)
        bits = lax.bitcast_convert_type(d, jnp.int32)
        y = lax.bitcast_convert_type(jnp.int32(0x5F3759DF) - (bits >> 1),
                                     jnp.float32)
        for _ in range(3):
            y = y * (jnp.float32(1.5) - jnp.float32(0.5) * d * y * y)
        disbuf[pl.ds(k * 16, 16)] = y
        return carry
    lax.fori_loop(0, ROWS_PER_TILE // 16, _rsqrt, 0)

    @pl.when(c == 0)
    def _():
        pltpu.sync_copy(disbuf, dis_out.at[pl.ds(row0, ROWS_PER_TILE)])

    for rc in range(ROWS_PER_TILE // ROWCHUNK):
        pltpu.sync_copy(h_hbm.at[pl.ds(row0 + rc * ROWCHUNK, ROWCHUNK)], hbuf)

        def _scale(j, carry):
            sc_vec = plsc.load_gather(
                disbuf, [jnp.full((16,), rc * ROWCHUNK, jnp.int32) + j])
            hbuf[j, :] = hbuf[j, :] * sc_vec
            return carry
        lax.fori_loop(0, ROWCHUNK, _scale, 0)
        pltpu.sync_copy(hbuf, h_sh.at[pl.ds(row0 + rc * ROWCHUNK, ROWCHUNK)])
    plsc.subcore_barrier()

    r0 = wid * CHUNKS_PER_TILE
    pltpu.sync_copy(src2d_hbm.at[pl.ds(r0, CHUNKS_PER_TILE)], srcidx)
    pltpu.sync_copy(dst2d_hbm.at[pl.ds(r0, CHUNKS_PER_TILE)], dstidx)

    pltpu.make_async_copy(h_sh.at[srcidx.at[0]], msga, sema).start()

    def _edges(t, carry):
        j0 = 2 * t
        pltpu.make_async_copy(h_sh.at[srcidx.at[j0 + 1]], msgb, semb).start()
        pltpu.make_async_copy(h_sh.at[srcidx.at[j0]], msga, sema).wait()
        pltpu.sync_copy(msga, acc_sh.at[dstidx.at[j0]], add=True)

        @pl.when(t < CHUNKS_PER_TILE // 2 - 1)
        def _():
            pltpu.make_async_copy(h_sh.at[srcidx.at[j0 + 2]], msga,
                                  sema).start()
        pltpu.make_async_copy(h_sh.at[srcidx.at[j0 + 1]], msgb, semb).wait()
        pltpu.sync_copy(msgb, acc_sh.at[dstidx.at[j0 + 1]], add=True)
        return carry
    lax.fori_loop(0, CHUNKS_PER_TILE // 2, _edges, 0)
    plsc.subcore_barrier()

    pltpu.sync_copy(acc_sh.at[pl.ds(row0, ROWS_PER_TILE)],
                    acc_out.at[c, pl.ds(row0, ROWS_PER_TILE)])


def _final_body(h_ref, acc_ref, dis_ref, batch_ref, b1_ref, w2_ref, b2_ref,
                o_ref):
    h = h_ref[...]
    acc = acc_ref[0] + acc_ref[1]
    dis = dis_ref[...]
    tmp = dis[:, None] * (dis[:, None] * h + acc) + b1_ref[...][None, :]
    nodes = jnp.maximum(tmp, 0.0)
    b = batch_ref[...]
    onehot = (b[:, None] == lax.broadcasted_iota(jnp.int32, (N_PAD, NG), 1)
              ).astype(jnp.float32)
    counts = jnp.sum(onehot, axis=0)
    sums = lax.dot_general(onehot, nodes, (((0,), (0,)), ((), ())),
                           preferred_element_type=jnp.float32)
    g = sums / jnp.clip(counts, 1.0)[:, None]
    logits = jnp.dot(g, w2_ref[...], preferred_element_type=jnp.float32)
    logits = logits + b2_ref[...][None, :]
    m = jnp.max(logits, axis=1, keepdims=True)
    lse = jnp.log(jnp.sum(jnp.exp(logits - m), axis=1, keepdims=True)) + m
    o_ref[...] = logits - lse


def _final(h, acc, dis, batch_pad, b1, W2, b2):
    return pl.pallas_call(
        _final_body,
        out_shape=jax.ShapeDtypeStruct((NG, 10), jnp.float32),
    )(h, acc, dis, batch_pad, b1, W2, b2)


def kernel(x, edge_index, batch, W1, b1, W2, b2):
    src = edge_index[0].astype(jnp.int32)
    dst = edge_index[1].astype(jnp.int32)
    pad = jnp.full((E_PAD - NUM_EDGES,), N_NODES, jnp.int32)
    src_p = jnp.concatenate([src, pad]).reshape(E_ROWS, CHUNK)
    dst_p = jnp.concatenate([dst, pad]).reshape(E_ROWS, CHUNK)
    x_pad = jnp.concatenate(
        [x, jnp.zeros((N_PAD - N_NODES, D_FEAT), jnp.float32)])
    batch_pad = jnp.concatenate(
        [batch.astype(jnp.int32), jnp.full((N_PAD - N_NODES,), NG, jnp.int32)])

    h = _matmul(x_pad, W1)
    acc, dis = _sc_gcn(h, src_p, dst_p)
    return _final(h, acc, dis, batch_pad, b1, W2, b2)


# speedup vs baseline: 60.7193x; 1.0083x over previous
import functools

import jax
import jax.numpy as jnp
from jax import lax
from jax.experimental import pallas as pl
from jax.experimental.pallas import tpu as pltpu
from jax.experimental.pallas import tpu_sc as plsc

N_NODES = 10000
N_PAD = 10240
ROWS_PER_TILE = 640
HID = 16
D_FEAT = 128
NUM_EDGES = 320000
CHUNK = 128
CHUNKS_PER_TILE = 80
E_ROWS = 32 * CHUNKS_PER_TILE
E_PAD = E_ROWS * CHUNK
HIST_ROWS_PER_TILE = E_ROWS // 16
HIST_ROWCHUNK = 16
ROWCHUNK = 160
NG = 64


def _mm_body(x_ref, w_ref, o_ref):
    o_ref[...] = jnp.dot(x_ref[...], w_ref[...],
                         preferred_element_type=jnp.float32)


def _matmul(x_pad, W1):
    return pl.pallas_call(
        _mm_body,
        out_shape=jax.ShapeDtypeStruct((N_PAD, HID), jnp.float32),
    )(x_pad, W1)


_MESH = plsc.VectorSubcoreMesh(core_axis_name="c", subcore_axis_name="s")


@functools.partial(
    pl.kernel,
    mesh=_MESH,
    compiler_params=pltpu.CompilerParams(needs_layout_passes=False,
                                         use_tc_tiling_on_sc=False),
    out_type=[
        jax.ShapeDtypeStruct((2, N_PAD, HID), jnp.float32),
        jax.ShapeDtypeStruct((N_PAD,), jnp.float32),
    ],
    scratch_types=[
        pltpu.VMEM((HIST_ROWCHUNK, CHUNK), jnp.int32),
        pltpu.VMEM((N_PAD,), jnp.float32),
        pltpu.VMEM((ROWS_PER_TILE,), jnp.float32),
        pltpu.VMEM((ROWCHUNK, HID), jnp.float32),
        pltpu.VMEM((CHUNKS_PER_TILE, CHUNK), jnp.int32),
        pltpu.VMEM((CHUNKS_PER_TILE, CHUNK), jnp.int32),
        pltpu.VMEM((CHUNK, HID), jnp.float32),
        pltpu.VMEM((CHUNK, HID), jnp.float32),
        pltpu.VMEM((CHUNK, HID), jnp.float32),
        pltpu.VMEM((CHUNK, HID), jnp.float32),
        pltpu.VMEM_SHARED((16, N_PAD), jnp.float32),
        pltpu.VMEM_SHARED((N_PAD, HID), jnp.float32),
        pltpu.VMEM_SHARED((N_PAD, HID), jnp.float32),
        pltpu.SemaphoreType.DMA,
        pltpu.SemaphoreType.DMA,
        pltpu.SemaphoreType.DMA,
        pltpu.SemaphoreType.DMA,
        pltpu.SemaphoreType.DMA,
        pltpu.SemaphoreType.DMA,
        pltpu.SemaphoreType.DMA,
        pltpu.SemaphoreType.DMA,
    ],
)
def _sc_gcn(h_hbm, src2d_hbm, dst2d_hbm,
            acc_out, dis_out,
            dstbuf, degbuf, disbuf, hbuf, srcidx, dstidx,
            msg0, msg1, msg2, msg3, stage_sh, h_sh, acc_sh,
            g0, g1, g2, g3, s0, s1, s2, s3):
    c = lax.axis_index("c")
    s = lax.axis_index("s")
    wid = c * 16 + s
    row0 = s * ROWS_PER_TILE

    zeros16 = jnp.zeros((16,), jnp.float32)
    ones16 = jnp.ones((16,), jnp.float32)

    def _zdeg(i, carry):
        degbuf[pl.ds(i * 16, 16)] = zeros16
        return carry
    lax.fori_loop(0, N_PAD // 16, _zdeg, 0)

    def _zh(i, carry):
        hbuf[i, :] = zeros16
        return carry
    lax.fori_loop(0, ROWCHUNK, _zh, 0)
    for rc in range(ROWS_PER_TILE // ROWCHUNK):
        pltpu.sync_copy(hbuf, acc_sh.at[pl.ds(row0 + rc * ROWCHUNK, ROWCHUNK)])

    for hc in range(HIST_ROWS_PER_TILE // HIST_ROWCHUNK):
        pltpu.sync_copy(
            dst2d_hbm.at[pl.ds(s * HIST_ROWS_PER_TILE + hc * HIST_ROWCHUNK,
                               HIST_ROWCHUNK)],
            dstbuf)

        def _histrow(r, carry):
            def _hist(k, carry2):
                idx = dstbuf[r, pl.ds(k * 16, 16)]
                plsc.addupdate_scatter(degbuf, [idx], ones16)
                return carry2
            lax.fori_loop(0, CHUNK // 16, _hist, 0)
            return carry
        lax.fori_loop(0, HIST_ROWCHUNK, _histrow, 0)
    pltpu.sync_copy(degbuf, stage_sh.at[s])
    plsc.subcore_barrier()

    pltpu.sync_copy(stage_sh.at[0, pl.ds(row0, ROWS_PER_TILE)], disbuf)
    for j in range(1, 16):
        pltpu.sync_copy(stage_sh.at[j, pl.ds(row0, ROWS_PER_TILE)],
                        degbuf.at[pl.ds(0, ROWS_PER_TILE)])

        def _acc(k, carry):
            disbuf[pl.ds(k * 16, 16)] = (disbuf[pl.ds(k * 16, 16)]
                                         + degbuf[pl.ds(k * 16, 16)])
            return carry
        lax.fori_loop(0, ROWS_PER_TILE // 16, _acc, 0)

    def _rsqrt(k, carry):
        d = disbuf[pl.ds(k * 16, 16)] + jnp.float32(1.0)
        bits = lax.bitcast_convert_type(d, jnp.int32)
        y = lax.bitcast_convert_type(jnp.int32(0x5F3759DF) - (bits >> 1),
                                     jnp.float32)
        for _ in range(3):
            y = y * (jnp.float32(1.5) - jnp.float32(0.5) * d * y * y)
        disbuf[pl.ds(k * 16, 16)] = y
        return carry
    lax.fori_loop(0, ROWS_PER_TILE // 16, _rsqrt, 0)

    @pl.when(c == 0)
    def _():
        pltpu.sync_copy(disbuf, dis_out.at[pl.ds(row0, ROWS_PER_TILE)])

    for rc in range(ROWS_PER_TILE // ROWCHUNK):
        pltpu.sync_copy(h_hbm.at[pl.ds(row0 + rc * ROWCHUNK, ROWCHUNK)], hbuf)

        def _scale(j, carry):
            sc_vec = plsc.load_gather(
                disbuf, [jnp.full((16,), rc * ROWCHUNK, jnp.int32) + j])
            hbuf[j, :] = hbuf[j, :] * sc_vec
            return carry
        lax.fori_loop(0, ROWCHUNK, _scale, 0)
        pltpu.sync_copy(hbuf, h_sh.at[pl.ds(row0 + rc * ROWCHUNK, ROWCHUNK)])
    plsc.subcore_barrier()

    r0 = wid * CHUNKS_PER_TILE
    pltpu.sync_copy(src2d_hbm.at[pl.ds(r0, CHUNKS_PER_TILE)], srcidx)
    pltpu.sync_copy(dst2d_hbm.at[pl.ds(r0, CHUNKS_PER_TILE)], dstidx)

    msgs = (msg0, msg1, msg2, msg3)
    gsem = (g0, g1, g2, g3)
    ssem = (s0, s1, s2, s3)

    def _gather(j, u):
        return pltpu.make_async_copy(h_sh.at[srcidx.at[j]], msgs[u], gsem[u])

    def _scatter(j, u):
        return pltpu.make_async_copy(msgs[u], acc_sh.at[dstidx.at[j]],
                                     ssem[u])

    _gather(0, 0).start()
    _gather(1, 1).start()

    def _edges(t, carry):
        for u in range(4):
            j = 4 * t + u
            v = (u + 2) % 4
            _gather(j, u).wait()
            _scatter(j, u).start(add=True)
            if u < 2:
                @pl.when(t > 0)
                def _():
                    _scatter(j - 2, v).wait()
                _gather(j + 2, v).start()
            else:
                _scatter(j - 2, v).wait()

                @pl.when(t < CHUNKS_PER_TILE // 4 - 1)
                def _():
                    _gather(j + 2, v).start()
        return carry
    lax.fori_loop(0, CHUNKS_PER_TILE // 4, _edges, 0)
    _scatter(CHUNKS_PER_TILE - 2, 2).wait()
    _scatter(CHUNKS_PER_TILE - 1, 3).wait()
    plsc.subcore_barrier()

    pltpu.sync_copy(acc_sh.at[pl.ds(row0, ROWS_PER_TILE)],
                    acc_out.at[c, pl.ds(row0, ROWS_PER_TILE)])


def _final_body(h_ref, acc_ref, dis_ref, batch_ref, b1_ref, w2_ref, b2_ref,
                o_ref):
    h = h_ref[...]
    acc = acc_ref[0] + acc_ref[1]
    dis = dis_ref[...]
    tmp = dis[:, None] * (dis[:, None] * h + acc) + b1_ref[...][None, :]
    nodes = jnp.maximum(tmp, 0.0)
    b = batch_ref[...]
    onehot = (b[:, None] == lax.broadcasted_iota(jnp.int32, (N_PAD, NG), 1)
              ).astype(jnp.float32)
    counts = jnp.sum(onehot, axis=0)
    sums = lax.dot_general(onehot, nodes, (((0,), (0,)), ((), ())),
                           preferred_element_type=jnp.float32)
    g = sums / jnp.clip(counts, 1.0)[:, None]
    logits = jnp.dot(g, w2_ref[...], preferred_element_type=jnp.float32)
    logits = logits + b2_ref[...][None, :]
    m = jnp.max(logits, axis=1, keepdims=True)
    lse = jnp.log(jnp.sum(jnp.exp(logits - m), axis=1, keepdims=True)) + m
    o_ref[...] = logits - lse


def _final(h, acc, dis, batch_pad, b1, W2, b2):
    return pl.pallas_call(
        _final_body,
        out_shape=jax.ShapeDtypeStruct((NG, 10), jnp.float32),
    )(h, acc, dis, batch_pad, b1, W2, b2)


def kernel(x, edge_index, batch, W1, b1, W2, b2):
    src = edge_index[0].astype(jnp.int32)
    dst = edge_index[1].astype(jnp.int32)
    pad = jnp.full((E_PAD - NUM_EDGES,), N_NODES, jnp.int32)
    src_p = jnp.concatenate([src, pad]).reshape(E_ROWS, CHUNK)
    dst_p = jnp.concatenate([dst, pad]).reshape(E_ROWS, CHUNK)
    x_pad = jnp.concatenate(
        [x, jnp.zeros((N_PAD - N_NODES, D_FEAT), jnp.float32)])
    batch_pad = jnp.concatenate(
        [batch.astype(jnp.int32), jnp.full((N_PAD - N_NODES,), NG, jnp.int32)])

    h = _matmul(x_pad, W1)
    acc, dis = _sc_gcn(h, src_p, dst_p)
    return _final(h, acc, dis, batch_pad, b1, W2, b2)


# speedup vs baseline: 60.7307x; 1.0002x over previous
import functools

import jax
import jax.numpy as jnp
from jax import lax
from jax.experimental import pallas as pl
from jax.experimental.pallas import tpu as pltpu
from jax.experimental.pallas import tpu_sc as plsc

N_NODES = 10000
N_PAD = 10240
ROWS_PER_TILE = 640
HID = 16
D_FEAT = 128
NUM_EDGES = 320000
CHUNK = 128
CHUNKS_PER_TILE = 80
E_ROWS = 32 * CHUNKS_PER_TILE
E_PAD = E_ROWS * CHUNK
HIST_ROWS_PER_TILE = E_ROWS // 16
HIST_ROWCHUNK = 16
ROWCHUNK = 160
NG = 64


def _mm_body(x_ref, w_ref, o_ref):
    o_ref[...] = jnp.dot(x_ref[...], w_ref[...],
                         preferred_element_type=jnp.float32)


def _matmul(x_pad, W1):
    return pl.pallas_call(
        _mm_body,
        out_shape=jax.ShapeDtypeStruct((N_PAD, HID), jnp.float32),
    )(x_pad, W1)


_MESH = plsc.VectorSubcoreMesh(core_axis_name="c", subcore_axis_name="s")


@functools.partial(
    pl.kernel,
    mesh=_MESH,
    compiler_params=pltpu.CompilerParams(needs_layout_passes=False,
                                         use_tc_tiling_on_sc=False),
    out_type=[
        jax.ShapeDtypeStruct((2, N_PAD, HID), jnp.float32),
        jax.ShapeDtypeStruct((N_PAD,), jnp.float32),
    ],
    scratch_types=[
        pltpu.VMEM((HIST_ROWCHUNK, CHUNK), jnp.int32),
        pltpu.VMEM((N_PAD,), jnp.float32),
        pltpu.VMEM((ROWS_PER_TILE,), jnp.float32),
        pltpu.VMEM((ROWCHUNK, HID), jnp.float32),
        pltpu.VMEM((CHUNKS_PER_TILE, CHUNK), jnp.int32),
        pltpu.VMEM((CHUNKS_PER_TILE, CHUNK), jnp.int32),
        pltpu.VMEM((CHUNK, HID), jnp.float32),
        pltpu.VMEM((CHUNK, HID), jnp.float32),
        pltpu.VMEM((CHUNK, HID), jnp.float32),
        pltpu.VMEM((CHUNK, HID), jnp.float32),
        pltpu.VMEM_SHARED((16, N_PAD), jnp.float32),
        pltpu.VMEM_SHARED((N_PAD, HID), jnp.float32),
        pltpu.VMEM_SHARED((N_PAD, HID), jnp.float32),
        pltpu.SemaphoreType.DMA,
        pltpu.SemaphoreType.DMA,
        pltpu.SemaphoreType.DMA,
        pltpu.SemaphoreType.DMA,
        pltpu.SemaphoreType.DMA,
        pltpu.SemaphoreType.DMA,
        pltpu.SemaphoreType.DMA,
        pltpu.SemaphoreType.DMA,
    ],
)
def _sc_gcn(h_hbm, src2d_hbm, dst2d_hbm,
            acc_out, dis_out,
            dstbuf, degbuf, disbuf, hbuf, srcidx, dstidx,
            msg0, msg1, msg2, msg3, stage_sh, h_sh, acc_sh,
            g0, g1, g2, g3, s0, s1, s2, s3):
    c = lax.axis_index("c")
    s = lax.axis_index("s")
    wid = c * 16 + s
    row0 = s * ROWS_PER_TILE

    zeros16 = jnp.zeros((16,), jnp.float32)
    ones16 = jnp.ones((16,), jnp.float32)

    def _zdeg(i, carry):
        degbuf[pl.ds(i * 16, 16)] = zeros16
        return carry
    lax.fori_loop(0, N_PAD // 16, _zdeg, 0)

    def _zh(i, carry):
        hbuf[i, :] = zeros16
        return carry
    lax.fori_loop(0, ROWCHUNK, _zh, 0)
    for rc in range(ROWS_PER_TILE // ROWCHUNK):
        pltpu.sync_copy(hbuf, acc_sh.at[pl.ds(row0 + rc * ROWCHUNK, ROWCHUNK)])

    for hc in range(HIST_ROWS_PER_TILE // HIST_ROWCHUNK):
        pltpu.sync_copy(
            dst2d_hbm.at[pl.ds(s * HIST_ROWS_PER_TILE + hc * HIST_ROWCHUNK,
                               HIST_ROWCHUNK)],
            dstbuf)

        def _histrow(r, carry):
            def _hist(k, carry2):
                idx = dstbuf[r, pl.ds(k * 16, 16)]
                plsc.addupdate_scatter(degbuf, [idx], ones16)
                return carry2
            lax.fori_loop(0, CHUNK // 16, _hist, 0)
            return carry
        lax.fori_loop(0, HIST_ROWCHUNK, _histrow, 0)
    pltpu.sync_copy(degbuf, stage_sh.at[s])
    plsc.subcore_barrier()

    pltpu.sync_copy(stage_sh.at[0, pl.ds(row0, ROWS_PER_TILE)], disbuf)
    for j in range(1, 16):
        pltpu.sync_copy(stage_sh.at[j, pl.ds(row0, ROWS_PER_TILE)],
                        degbuf.at[pl.ds(0, ROWS_PER_TILE)])

        def _acc(k, carry):
            disbuf[pl.ds(k * 16, 16)] = (disbuf[pl.ds(k * 16, 16)]
                                         + degbuf[pl.ds(k * 16, 16)])
            return carry
        lax.fori_loop(0, ROWS_PER_TILE // 16, _acc, 0)

    def _rsqrt(k, carry):
        d = disbuf[pl.ds(k * 16, 16)] + jnp.float32(1.0)
        bits = lax.bitcast_convert_type(d, jnp.int32)
        y = lax.bitcast_convert_type(jnp.int32(0x5F3759DF) - (bits >> 1),
                                     jnp.float32)
        for _ in range(3):
            y = y * (jnp.float32(1.5) - jnp.float32(0.5) * d * y * y)
        disbuf[pl.ds(k * 16, 16)] = y
        return carry
    lax.fori_loop(0, ROWS_PER_TILE // 16, _rsqrt, 0)

    @pl.when(c == 0)
    def _():
        pltpu.sync_copy(disbuf, dis_out.at[pl.ds(row0, ROWS_PER_TILE)])

    for rc in range(ROWS_PER_TILE // ROWCHUNK):
        pltpu.sync_copy(h_hbm.at[pl.ds(row0 + rc * ROWCHUNK, ROWCHUNK)], hbuf)

        def _scale(j, carry):
            sc_vec = plsc.load_gather(
                disbuf, [jnp.full((16,), rc * ROWCHUNK, jnp.int32) + j])
            hbuf[j, :] = hbuf[j, :] * sc_vec
            return carry
        lax.fori_loop(0, ROWCHUNK, _scale, 0)
        pltpu.sync_copy(hbuf, h_sh.at[pl.ds(row0 + rc * ROWCHUNK, ROWCHUNK)])
    plsc.subcore_barrier()

    r0 = wid * CHUNKS_PER_TILE
    pltpu.sync_copy(src2d_hbm.at[pl.ds(r0, CHUNKS_PER_TILE)], srcidx)
    pltpu.sync_copy(dst2d_hbm.at[pl.ds(r0, CHUNKS_PER_TILE)], dstidx)

    msgs = (msg0, msg1, msg2, msg3)
    gsem = (g0, g1, g2, g3)
    ssem = (s0, s1, s2, s3)

    def _gather(j, u):
        return pltpu.make_async_copy(h_sh.at[srcidx.at[j]], msgs[u], gsem[u])

    def _scatter(j, u):
        return pltpu.make_async_copy(msgs[u], acc_sh.at[dstidx.at[j]],
                                     ssem[u])

    _gather(0, 0).start()
    _gather(1, 1).start()
    for u in range(4):
        _gather(u, u).wait()
        _scatter(u, u).start(add=True)
        if u >= 2:
            _scatter(u - 2, u - 2).wait()
        _gather(u + 2, (u + 2) % 4).start()

    def _edges(t, carry):
        for u in range(4):
            j = 4 * t + u
            v = (u + 2) % 4
            _gather(j, u).wait()
            _scatter(j, u).start(add=True)
            _scatter(j - 2, v).wait()
            _gather(j + 2, v).start()
        return carry
    lax.fori_loop(1, CHUNKS_PER_TILE // 4 - 1, _edges, 0)

    for u in range(4):
        j = CHUNKS_PER_TILE - 4 + u
        v = (u + 2) % 4
        _gather(j, u).wait()
        _scatter(j, u).start(add=True)
        _scatter(j - 2, v).wait()
        if u < 2:
            _gather(j + 2, v).start()
    _scatter(CHUNKS_PER_TILE - 2, 2).wait()
    _scatter(CHUNKS_PER_TILE - 1, 3).wait()
    plsc.subcore_barrier()

    pltpu.sync_copy(acc_sh.at[pl.ds(row0, ROWS_PER_TILE)],
                    acc_out.at[c, pl.ds(row0, ROWS_PER_TILE)])


def _final_body(h_ref, acc_ref, dis_ref, batch_ref, b1_ref, w2_ref, b2_ref,
                o_ref):
    h = h_ref[...]
    acc = acc_ref[0] + acc_ref[1]
    dis = dis_ref[...]
    tmp = dis[:, None] * (dis[:, None] * h + acc) + b1_ref[...][None, :]
    nodes = jnp.maximum(tmp, 0.0)
    b = batch_ref[...]
    onehot = (b[:, None] == lax.broadcasted_iota(jnp.int32, (N_PAD, NG), 1)
              ).astype(jnp.float32)
    counts = jnp.sum(onehot, axis=0)
    sums = lax.dot_general(onehot, nodes, (((0,), (0,)), ((), ())),
                           preferred_element_type=jnp.float32)
    g = sums / jnp.clip(counts, 1.0)[:, None]
    logits = jnp.dot(g, w2_ref[...], preferred_element_type=jnp.float32)
    logits = logits + b2_ref[...][None, :]
    m = jnp.max(logits, axis=1, keepdims=True)
    lse = jnp.log(jnp.sum(jnp.exp(logits - m), axis=1, keepdims=True)) + m
    o_ref[...] = logits - lse


def _final(h, acc, dis, batch_pad, b1, W2, b2):
    return pl.pallas_call(
        _final_body,
        out_shape=jax.ShapeDtypeStruct((NG, 10), jnp.float32),
    )(h, acc, dis, batch_pad, b1, W2, b2)


def kernel(x, edge_index, batch, W1, b1, W2, b2):
    src = edge_index[0].astype(jnp.int32)
    dst = edge_index[1].astype(jnp.int32)
    pad = jnp.full((E_PAD - NUM_EDGES,), N_NODES, jnp.int32)
    src_p = jnp.concatenate([src, pad]).reshape(E_ROWS, CHUNK)
    dst_p = jnp.concatenate([dst, pad]).reshape(E_ROWS, CHUNK)
    x_pad = jnp.concatenate(
        [x, jnp.zeros((N_PAD - N_NODES, D_FEAT), jnp.float32)])
    batch_pad = jnp.concatenate(
        [batch.astype(jnp.int32), jnp.full((N_PAD - N_NODES,), NG, jnp.int32)])

    h = _matmul(x_pad, W1)
    acc, dis = _sc_gcn(h, src_p, dst_p)
    return _final(h, acc, dis, batch_pad, b1, W2, b2)


# speedup vs baseline: 62.6941x; 1.0323x over previous
import functools

import jax
import jax.numpy as jnp
from jax import lax
from jax.experimental import pallas as pl
from jax.experimental.pallas import tpu as pltpu
from jax.experimental.pallas import tpu_sc as plsc

N_NODES = 10000
N_PAD = 10240
ROWS_PER_TILE = 640
HID = 16
D_FEAT = 128
NUM_EDGES = 320000
CHUNK = 128
CHUNKS_PER_TILE = 80
E_ROWS = 32 * CHUNKS_PER_TILE
E_PAD = E_ROWS * CHUNK
HIST_ROWS_PER_TILE = E_ROWS // 16
HIST_ROWCHUNK = 16
ROWCHUNK = 160
NG = 64


def _mm_body(x_ref, w_ref, o_ref):
    o_ref[pl.ds(0, N_NODES), :] = jnp.dot(x_ref[...], w_ref[...],
                                          preferred_element_type=jnp.float32)
    o_ref[pl.ds(N_NODES, N_PAD - N_NODES), :] = jnp.zeros(
        (N_PAD - N_NODES, HID), jnp.float32)


def _matmul(x, W1):
    return pl.pallas_call(
        _mm_body,
        out_shape=jax.ShapeDtypeStruct((N_PAD, HID), jnp.float32),
    )(x, W1)


_MESH = plsc.VectorSubcoreMesh(core_axis_name="c", subcore_axis_name="s")


@functools.partial(
    pl.kernel,
    mesh=_MESH,
    compiler_params=pltpu.CompilerParams(needs_layout_passes=False,
                                         use_tc_tiling_on_sc=False),
    out_type=[
        jax.ShapeDtypeStruct((2, N_PAD, HID), jnp.float32),
        jax.ShapeDtypeStruct((N_PAD,), jnp.float32),
    ],
    scratch_types=[
        pltpu.VMEM((HIST_ROWCHUNK, CHUNK), jnp.int32),
        pltpu.VMEM((N_PAD,), jnp.float32),
        pltpu.VMEM((ROWS_PER_TILE,), jnp.float32),
        pltpu.VMEM((ROWCHUNK, HID), jnp.float32),
        pltpu.VMEM((CHUNKS_PER_TILE, CHUNK), jnp.int32),
        pltpu.VMEM((CHUNKS_PER_TILE, CHUNK), jnp.int32),
        pltpu.VMEM((CHUNK, HID), jnp.float32),
        pltpu.VMEM((CHUNK, HID), jnp.float32),
        pltpu.VMEM((CHUNK, HID), jnp.float32),
        pltpu.VMEM((CHUNK, HID), jnp.float32),
        pltpu.VMEM((CHUNK, HID), jnp.float32),
        pltpu.VMEM((CHUNK, HID), jnp.float32),
        pltpu.VMEM((CHUNK, HID), jnp.float32),
        pltpu.VMEM((CHUNK, HID), jnp.float32),
        pltpu.VMEM_SHARED((16, N_PAD), jnp.float32),
        pltpu.VMEM_SHARED((N_PAD, HID), jnp.float32),
        pltpu.VMEM_SHARED((N_PAD, HID), jnp.float32),
        pltpu.SemaphoreType.DMA,
        pltpu.SemaphoreType.DMA,
        pltpu.SemaphoreType.DMA,
        pltpu.SemaphoreType.DMA,
        pltpu.SemaphoreType.DMA,
        pltpu.SemaphoreType.DMA,
        pltpu.SemaphoreType.DMA,
        pltpu.SemaphoreType.DMA,
        pltpu.SemaphoreType.DMA,
        pltpu.SemaphoreType.DMA,
        pltpu.SemaphoreType.DMA,
        pltpu.SemaphoreType.DMA,
        pltpu.SemaphoreType.DMA,
        pltpu.SemaphoreType.DMA,
        pltpu.SemaphoreType.DMA,
        pltpu.SemaphoreType.DMA,
    ],
)
def _sc_gcn(h_hbm, src2d_hbm, dst2d_hbm,
            acc_out, dis_out,
            dstbuf, degbuf, disbuf, hbuf, srcidx, dstidx,
            msg0, msg1, msg2, msg3, msg4, msg5, msg6, msg7,
            stage_sh, h_sh, acc_sh,
            g0, g1, g2, g3, g4, g5, g6, g7,
            s0, s1, s2, s3, s4, s5, s6, s7):
    c = lax.axis_index("c")
    s = lax.axis_index("s")
    wid = c * 16 + s
    row0 = s * ROWS_PER_TILE

    zeros16 = jnp.zeros((16,), jnp.float32)
    ones16 = jnp.ones((16,), jnp.float32)

    def _zdeg(i, carry):
        degbuf[pl.ds(i * 16, 16)] = zeros16
        return carry
    lax.fori_loop(0, N_PAD // 16, _zdeg, 0)

    def _zh(i, carry):
        hbuf[i, :] = zeros16
        return carry
    lax.fori_loop(0, ROWCHUNK, _zh, 0)
    for rc in range(ROWS_PER_TILE // ROWCHUNK):
        pltpu.sync_copy(hbuf, acc_sh.at[pl.ds(row0 + rc * ROWCHUNK, ROWCHUNK)])

    for hc in range(HIST_ROWS_PER_TILE // HIST_ROWCHUNK):
        pltpu.sync_copy(
            dst2d_hbm.at[pl.ds(s * HIST_ROWS_PER_TILE + hc * HIST_ROWCHUNK,
                               HIST_ROWCHUNK)],
            dstbuf)

        def _histrow(r, carry):
            def _hist(k, carry2):
                idx = dstbuf[r, pl.ds(k * 16, 16)]
                plsc.addupdate_scatter(degbuf, [idx], ones16)
                return carry2
            lax.fori_loop(0, CHUNK // 16, _hist, 0)
            return carry
        lax.fori_loop(0, HIST_ROWCHUNK, _histrow, 0)
    pltpu.sync_copy(degbuf, stage_sh.at[s])
    plsc.subcore_barrier()

    pltpu.sync_copy(stage_sh.at[0, pl.ds(row0, ROWS_PER_TILE)], disbuf)
    for j in range(1, 16):
        pltpu.sync_copy(stage_sh.at[j, pl.ds(row0, ROWS_PER_TILE)],
                        degbuf.at[pl.ds(0, ROWS_PER_TILE)])

        def _acc(k, carry):
            disbuf[pl.ds(k * 16, 16)] = (disbuf[pl.ds(k * 16, 16)]
                                         + degbuf[pl.ds(k * 16, 16)])
            return carry
        lax.fori_loop(0, ROWS_PER_TILE // 16, _acc, 0)

    def _rsqrt(k, carry):
        d = disbuf[pl.ds(k * 16, 16)] + jnp.float32(1.0)
        bits = lax.bitcast_convert_type(d, jnp.int32)
        y = lax.bitcast_convert_type(jnp.int32(0x5F3759DF) - (bits >> 1),
                                     jnp.float32)
        for _ in range(3):
            y = y * (jnp.float32(1.5) - jnp.float32(0.5) * d * y * y)
        disbuf[pl.ds(k * 16, 16)] = y
        return carry
    lax.fori_loop(0, ROWS_PER_TILE // 16, _rsqrt, 0)

    @pl.when(c == 0)
    def _():
        pltpu.sync_copy(disbuf, dis_out.at[pl.ds(row0, ROWS_PER_TILE)])

    for rc in range(ROWS_PER_TILE // ROWCHUNK):
        pltpu.sync_copy(h_hbm.at[pl.ds(row0 + rc * ROWCHUNK, ROWCHUNK)], hbuf)

        def _scale(j, carry):
            sc_vec = plsc.load_gather(
                disbuf, [jnp.full((16,), rc * ROWCHUNK, jnp.int32) + j])
            hbuf[j, :] = hbuf[j, :] * sc_vec
            return carry
        lax.fori_loop(0, ROWCHUNK, _scale, 0)
        pltpu.sync_copy(hbuf, h_sh.at[pl.ds(row0 + rc * ROWCHUNK, ROWCHUNK)])
    plsc.subcore_barrier()

    r0 = wid * CHUNKS_PER_TILE
    pltpu.sync_copy(src2d_hbm.at[pl.ds(r0, CHUNKS_PER_TILE)], srcidx)
    pltpu.sync_copy(dst2d_hbm.at[pl.ds(r0, CHUNKS_PER_TILE)], dstidx)

    msgs = (msg0, msg1, msg2, msg3, msg4, msg5, msg6, msg7)
    gsem = (g0, g1, g2, g3, g4, g5, g6, g7)
    ssem = (s0, s1, s2, s3, s4, s5, s6, s7)

    def _gather(j, u):
        return pltpu.make_async_copy(h_sh.at[srcidx.at[j]], msgs[u], gsem[u])

    def _scatter(j, u):
        return pltpu.make_async_copy(msgs[u], acc_sh.at[dstidx.at[j]],
                                     ssem[u])

    for u in range(4):
        _gather(u, u).start()
    for u in range(8):
        _gather(u, u).wait()
        _scatter(u, u).start(add=True)
        if u >= 4:
            _scatter(u - 4, u - 4).wait()
        _gather(u + 4, (u + 4) % 8).start()

    def _edges(t, carry):
        for u in range(8):
            j = 8 * t + u
            v = (u + 4) % 8
            _gather(j, u).wait()
            _scatter(j, u).start(add=True)
            _scatter(j - 4, v).wait()
            _gather(j + 4, v).start()
        return carry
    lax.fori_loop(1, CHUNKS_PER_TILE // 8 - 1, _edges, 0)

    for u in range(8):
        j = CHUNKS_PER_TILE - 8 + u
        v = (u + 4) % 8
        _gather(j, u).wait()
        _scatter(j, u).start(add=True)
        _scatter(j - 4, v).wait()
        if u < 4:
            _gather(j + 4, v).start()
    for u in range(4, 8):
        _scatter(CHUNKS_PER_TILE - 8 + u, u).wait()
    plsc.subcore_barrier()

    pltpu.sync_copy(acc_sh.at[pl.ds(row0, ROWS_PER_TILE)],
                    acc_out.at[c, pl.ds(row0, ROWS_PER_TILE)])


def _final_body(h_ref, acc_ref, dis_ref, batch_ref, b1_ref, w2_ref, b2_ref,
                o_ref):
    h = h_ref[...]
    acc = acc_ref[0] + acc_ref[1]
    dis = dis_ref[...]
    tmp = dis[:, None] * (dis[:, None] * h + acc) + b1_ref[...][None, :]
    nodes = jnp.maximum(tmp, 0.0)
    b = batch_ref[...]
    onehot = (b[:, None] == lax.broadcasted_iota(jnp.int32, (N_PAD, NG), 1)
              ).astype(jnp.float32)
    counts = jnp.sum(onehot, axis=0)
    sums = lax.dot_general(onehot, nodes, (((0,), (0,)), ((), ())),
                           preferred_element_type=jnp.float32)
    g = sums / jnp.clip(counts, 1.0)[:, None]
    logits = jnp.dot(g, w2_ref[...], preferred_element_type=jnp.float32)
    logits = logits + b2_ref[...][None, :]
    m = jnp.max(logits, axis=1, keepdims=True)
    lse = jnp.log(jnp.sum(jnp.exp(logits - m), axis=1, keepdims=True)) + m
    o_ref[...] = logits - lse


def _final(h, acc, dis, batch_pad, b1, W2, b2):
    return pl.pallas_call(
        _final_body,
        out_shape=jax.ShapeDtypeStruct((NG, 10), jnp.float32),
    )(h, acc, dis, batch_pad, b1, W2, b2)


def kernel(x, edge_index, batch, W1, b1, W2, b2):
    src = edge_index[0].astype(jnp.int32)
    dst = edge_index[1].astype(jnp.int32)
    pad = jnp.full((E_PAD - NUM_EDGES,), N_NODES, jnp.int32)
    src_p = jnp.concatenate([src, pad]).reshape(E_ROWS, CHUNK)
    dst_p = jnp.concatenate([dst, pad]).reshape(E_ROWS, CHUNK)
    batch_pad = jnp.concatenate(
        [batch.astype(jnp.int32), jnp.full((N_PAD - N_NODES,), NG, jnp.int32)])

    h = _matmul(x, W1)
    acc, dis = _sc_gcn(h, src_p, dst_p)
    return _final(h, acc, dis, batch_pad, b1, W2, b2)
